# Initial kernel scaffold; baseline (speedup 1.0000x reference)
import jax
import jax.numpy as jnp
from jax.experimental import pallas as pl


def kernel(x, edge_index, data, Wl0, bl0, Wr0, Wl1, bl1, Wr1, W1, b1, W2, b2):
    raise NotImplementedError("write your pallas kernel here")


# speedup vs baseline: 4.9296x; 4.9296x over previous
import jax
import jax.numpy as jnp
from jax import lax
from jax.experimental import pallas as pl
from jax.experimental.pallas import tpu as pltpu
from jax.experimental.pallas import tpu_sc as plsc

N = 10000
E = 160000
D = 256
H = 128
NC = 2
NS = 16
C = 80
E_PER_TILE = E // NS
NCHUNK = E_PER_TILE // C
STRIPE = 624
TAIL = N - NS * STRIPE
ZROWS = 16


def _sc_agg_body(h_hbm, src_hbm, dst_hbm, out_hbm,
                 src1d, dst1d, adj, dstb, rows, zbuf, acc, gsem):
    c = lax.axis_index("c")
    s = lax.axis_index("s")

    def _zrow(i, _):
        for j in range(H // 16):
            zbuf[i, pl.ds(j * 16, 16)] = jnp.zeros((16,), jnp.float32)
        return 0
    lax.fori_loop(0, ZROWS, _zrow, 0)

    stripe = s * STRIPE
    last = s == NS - 1

    def _zcp(j, _):
        pltpu.sync_copy(zbuf, acc.at[pl.ds(stripe + j * ZROWS, ZROWS)])
        return 0
    lax.fori_loop(0, STRIPE // ZROWS, _zcp, 0)

    @pl.when(last)
    def _():
        pltpu.sync_copy(zbuf, acc.at[pl.ds(NS * STRIPE, TAIL)])

    pltpu.sync_copy(src_hbm.at[pl.ds(s * E_PER_TILE, E_PER_TILE)], src1d)
    pltpu.sync_copy(dst_hbm.at[pl.ds(s * E_PER_TILE, E_PER_TILE)], dst1d)

    plsc.subcore_barrier()

    half_off = c * N

    def _chunk(t, _):
        for j in range(C // 16):
            sl = pl.ds(j * 16, 16)
            esl = pl.ds(t * C + j * 16, 16)
            adj[sl] = src1d[esl] + half_off
            dstb[sl] = dst1d[esl]
        pltpu.async_copy(h_hbm.at[adj], rows, gsem).wait()
        pltpu.sync_copy(rows, acc.at[dstb], add=True)
        return 0

    lax.fori_loop(0, NCHUNK, _chunk, 0)

    plsc.subcore_barrier()

    pltpu.sync_copy(acc.at[pl.ds(stripe, STRIPE)],
                    out_hbm.at[pl.ds(c * N + stripe, STRIPE)])

    @pl.when(last)
    def _():
        pltpu.sync_copy(acc.at[pl.ds(NS * STRIPE, TAIL)],
                        out_hbm.at[pl.ds(c * N + NS * STRIPE, TAIL)])


def _sc_aggregate(h_stacked, src, dst):
    mesh = plsc.VectorSubcoreMesh(core_axis_name="c", subcore_axis_name="s")
    k = pl.kernel(
        _sc_agg_body,
        out_type=jax.ShapeDtypeStruct((NC * N, H), jnp.float32),
        mesh=mesh,
        scratch_types=(
            pltpu.VMEM((E_PER_TILE,), jnp.int32),
            pltpu.VMEM((E_PER_TILE,), jnp.int32),
            pltpu.VMEM((C,), jnp.int32),
            pltpu.VMEM((C,), jnp.int32),
            pltpu.VMEM((C, H), jnp.float32),
            pltpu.VMEM((ZROWS, H), jnp.float32),
            pltpu.VMEM_SHARED((N, H), jnp.float32),
            pltpu.SemaphoreType.DMA,
        ),
    )
    return k(h_stacked, src, dst)


E_PER_W = 5040
E_PAD = E_PER_W * NC * NS
NCHUNK_CNT = E_PER_W // C


def _sc_cnt_body(dst_hbm, cnt_hbm, dst1d, dstb, onesb, zbuf, cntacc):
    c = lax.axis_index("c")
    s = lax.axis_index("s")

    def _zrow(i, _):
        for j in range(H // 16):
            zbuf[i, pl.ds(j * 16, 16)] = jnp.zeros((16,), jnp.float32)
        return 0
    lax.fori_loop(0, ZROWS, _zrow, 0)

    def _ones(i, _):
        for j in range(H // 16):
            onesb[i, pl.ds(j * 16, 16)] = jnp.ones((16,), jnp.float32)
        return 0
    lax.fori_loop(0, C, _ones, 0)

    stripe = s * STRIPE
    last = s == NS - 1

    def _zcp(j, _):
        pltpu.sync_copy(zbuf, cntacc.at[pl.ds(stripe + j * ZROWS, ZROWS)])
        return 0
    lax.fori_loop(0, STRIPE // ZROWS, _zcp, 0)

    @pl.when(last)
    def _():
        pltpu.sync_copy(zbuf, cntacc.at[pl.ds(NS * STRIPE, TAIL)])

    w = c * NS + s
    pltpu.sync_copy(dst_hbm.at[pl.ds(w * E_PER_W, E_PER_W)], dst1d)

    plsc.subcore_barrier()

    def _chunk(t, _):
        for j in range(C // 16):
            dstb[pl.ds(j * 16, 16)] = dst1d[pl.ds(t * C + j * 16, 16)]
        pltpu.sync_copy(onesb, cntacc.at[dstb], add=True)
        return 0

    lax.fori_loop(0, NCHUNK_CNT, _chunk, 0)

    plsc.subcore_barrier()

    pltpu.sync_copy(cntacc.at[pl.ds(stripe, STRIPE)],
                    cnt_hbm.at[c, pl.ds(stripe, STRIPE)])

    @pl.when(last)
    def _():
        pltpu.sync_copy(cntacc.at[pl.ds(NS * STRIPE, TAIL)],
                        cnt_hbm.at[c, pl.ds(NS * STRIPE, TAIL)])


def _sc_counts(dst_pad):
    mesh = plsc.VectorSubcoreMesh(core_axis_name="c", subcore_axis_name="s")
    k = pl.kernel(
        _sc_cnt_body,
        out_type=jax.ShapeDtypeStruct((NC, N, H), jnp.float32),
        mesh=mesh,
        scratch_types=(
            pltpu.VMEM((E_PER_W,), jnp.int32),
            pltpu.VMEM((C,), jnp.int32),
            pltpu.VMEM((C, H), jnp.float32),
            pltpu.VMEM((ZROWS, H), jnp.float32),
            pltpu.VMEM_SHARED((N + 16, H), jnp.float32),
        ),
    )
    return k(dst_pad)


BT = 1000


def _dense0_body(s_ref, cnt_ref, x_ref, wl_ref, bl_ref, wr_ref,
                 data_ref, w1_ref, b1_ref, w2_ref, b2_ref,
                 hn_ref, st_ref, mlp_ref):
    summed = jnp.concatenate([s_ref[0], s_ref[1]], axis=-1)
    cnt = cnt_ref[0, :, 0:1] + cnt_ref[1, :, 0:1]
    inv = 1.0 / jnp.maximum(cnt, 1.0)
    mean = summed * inv
    out = (jnp.dot(mean, wl_ref[:], preferred_element_type=jnp.float32)
           + bl_ref[0]
           + jnp.dot(x_ref[:], wr_ref[:], preferred_element_type=jnp.float32))
    nrm = jnp.sqrt(jnp.sum(out * out, axis=-1, keepdims=True))
    hn = out / jnp.maximum(nrm, 1e-12)
    hn_ref[:] = hn
    st_ref[0] = hn[:, :H]
    st_ref[1] = hn[:, H:]
    hid = jnp.maximum(
        jnp.dot(data_ref[:], w1_ref[:], preferred_element_type=jnp.float32)
        + b1_ref[0], 0.0)
    mlp_ref[:] = jax.nn.sigmoid(
        jnp.dot(hid, w2_ref[:], preferred_element_type=jnp.float32)
        + b2_ref[0])


def _dense1_body(s_ref, cnt_ref, x_ref, wl_ref, bl_ref, wr_ref, hn_ref):
    summed = jnp.concatenate([s_ref[0], s_ref[1]], axis=-1)
    cnt = cnt_ref[0, :, 0:1] + cnt_ref[1, :, 0:1]
    inv = 1.0 / jnp.maximum(cnt, 1.0)
    mean = summed * inv
    out = (jnp.dot(mean, wl_ref[:], preferred_element_type=jnp.float32)
           + bl_ref[0]
           + jnp.dot(x_ref[:], wr_ref[:], preferred_element_type=jnp.float32))
    nrm = jnp.sqrt(jnp.sum(out * out, axis=-1, keepdims=True))
    hn_ref[:] = out / jnp.maximum(nrm, 1e-12)


def _row_spec(shape):
    nd = len(shape)
    if nd == 2:
        return pl.BlockSpec((BT,) + shape[1:], lambda i: (i,) + (0,) * (nd - 1))
    return pl.BlockSpec((shape[0], BT) + shape[2:], lambda i: (0, i) + (0,) * (nd - 2))


def _full_spec(shape):
    nd = len(shape)
    return pl.BlockSpec(shape, lambda i: (0,) * nd)


def _tc_dense0(summed, cnt, x, wlT, bl, wrT, data, w1T, b1, w2T, b2):
    grid = (N // BT,)
    return pl.pallas_call(
        _dense0_body,
        grid=grid,
        in_specs=[
            _row_spec((2, N, H)), _row_spec((2, N, H)), _row_spec((N, D)),
            _full_spec((D, D)), _full_spec((1, D)), _full_spec((D, D)),
            _row_spec((N, D)), _full_spec((D, 32)), _full_spec((1, 32)),
            _full_spec((32, 2)), _full_spec((1, 2)),
        ],
        out_specs=[_row_spec((N, D)), _row_spec((2, N, H)), _row_spec((N, 2))],
        out_shape=[
            jax.ShapeDtypeStruct((N, D), jnp.float32),
            jax.ShapeDtypeStruct((2, N, H), jnp.float32),
            jax.ShapeDtypeStruct((N, 2), jnp.float32),
        ],
    )(summed, cnt, x, wlT, bl, wrT, data, w1T, b1, w2T, b2)


def _tc_dense1(summed, cnt, h, wlT, bl, wrT):
    grid = (N // BT,)
    return pl.pallas_call(
        _dense1_body,
        grid=grid,
        in_specs=[
            _row_spec((2, N, H)), _row_spec((2, N, H)), _row_spec((N, D)),
            _full_spec((D, D)), _full_spec((1, D)), _full_spec((D, D)),
        ],
        out_specs=[_row_spec((N, D))],
        out_shape=[jax.ShapeDtypeStruct((N, D), jnp.float32)],
    )(summed, cnt, h, wlT, bl, wrT)


def kernel(x, edge_index, data, Wl0, bl0, Wr0, Wl1, bl1, Wr1, W1, b1, W2, b2):
    src = edge_index[0]
    dst = edge_index[1]
    x_st = jnp.concatenate([x[:, :H], x[:, H:]], axis=0)

    dst_pad = jnp.concatenate(
        [dst, jnp.full((E_PAD - E,), N, jnp.int32)])
    cnt = _sc_counts(dst_pad)
    summed0 = _sc_aggregate(x_st, src, dst)
    h0, h0_st, mlp_out = _tc_dense0(
        summed0.reshape(2, N, H), cnt, x,
        Wl0.T, bl0.reshape(1, D), Wr0.T,
        data, W1.T, b1.reshape(1, 32), W2.T, b2.reshape(1, 2))

    summed1 = _sc_aggregate(h0_st.reshape(2 * N, H), src, dst)
    w_pred = _tc_dense1(summed1.reshape(2, N, H), cnt,
                        h0, Wl1.T, bl1.reshape(1, D), Wr1.T)[0]
    return (w_pred, mlp_out)


# speedup vs baseline: 6.9522x; 1.4103x over previous
import jax
import jax.numpy as jnp
from jax import lax
from jax.experimental import pallas as pl
from jax.experimental.pallas import tpu as pltpu
from jax.experimental.pallas import tpu_sc as plsc

N = 10000
E = 160000
D = 256
H = 128
NC = 2
NS = 16
C = 64
EPT = E // NS
NRING = 156
ETAIL = EPT - NRING * C
STRIPE = 624
TAIL = N - NS * STRIPE
ZROWS = 16
ZROWSA = 8


def _sc_agg_body(h_hbm, src2_hbm, dst_hbm, out_hbm,
                 src1d, dst1d, dstb0, dstb1,
                 rows0, rows1, dstbt, zbuf, acc, sem0, sem1):
    c = lax.axis_index("c")
    s = lax.axis_index("s")
    dsts, rows, sems = (dstb0, dstb1), (rows0, rows1), (sem0, sem1)

    def _fill(t, b):
        for j in range(C // 16):
            dsts[b][pl.ds(j * 16, 16)] = dst1d[pl.ds(t * C + j * 16, 16)]

    def _fire(t, b):
        pltpu.async_copy(h_hbm.at[src1d.at[pl.ds(t * C, C)]], rows[b], sems[b])

    pltpu.sync_copy(src2_hbm.at[pl.ds(c * E + s * EPT, EPT)], src1d)
    pltpu.sync_copy(dst_hbm.at[pl.ds(s * EPT, EPT)], dst1d)
    for b in range(2):
        _fill(b, b)
        _fire(b, b)

    def _zrow(i, _):
        for j in range(H // 16):
            zbuf[i, pl.ds(j * 16, 16)] = jnp.zeros((16,), jnp.float32)
        return 0
    lax.fori_loop(0, ZROWSA, _zrow, 0)

    stripe = s * STRIPE
    last = s == NS - 1

    def _zcp(j, _):
        pltpu.sync_copy(zbuf, acc.at[pl.ds(stripe + j * ZROWSA, ZROWSA)])
        return 0
    lax.fori_loop(0, STRIPE // ZROWSA, _zcp, 0)

    @pl.when(last)
    def _():
        pltpu.sync_copy(zbuf, acc.at[pl.ds(NS * STRIPE, ZROWSA)])
        pltpu.sync_copy(zbuf, acc.at[pl.ds(NS * STRIPE + ZROWSA, ZROWSA)])

    plsc.subcore_barrier()

    def _group(g, _):
        for b in range(2):
            pltpu.make_async_copy(
                h_hbm.at[src1d.at[pl.ds(0, C)]], rows[b], sems[b]).wait()
            pltpu.sync_copy(rows[b], acc.at[dsts[b]], add=True)
            t = g * 2 + b + 2
            _fill(t, b)
            _fire(t, b)
        return 0

    lax.fori_loop(0, (NRING - 2) // 2, _group, 0)

    for b in range(2):
        pltpu.make_async_copy(
            h_hbm.at[src1d.at[pl.ds(0, C)]], rows[b], sems[b]).wait()
        pltpu.sync_copy(rows[b], acc.at[dsts[b]], add=True)

    dstbt[:] = dst1d[pl.ds(NRING * C, ETAIL)]
    pltpu.async_copy(h_hbm.at[src1d.at[pl.ds(NRING * C, ETAIL)]],
                     rows0.at[pl.ds(0, ETAIL)], sem0)
    pltpu.make_async_copy(h_hbm.at[src1d.at[pl.ds(NRING * C, ETAIL)]],
                          rows0.at[pl.ds(0, ETAIL)], sem0).wait()
    pltpu.sync_copy(rows0.at[pl.ds(0, ETAIL)], acc.at[dstbt], add=True)

    plsc.subcore_barrier()

    pltpu.sync_copy(acc.at[pl.ds(stripe, STRIPE)],
                    out_hbm.at[pl.ds(c * N + stripe, STRIPE)])

    @pl.when(last)
    def _():
        pltpu.sync_copy(acc.at[pl.ds(NS * STRIPE, TAIL)],
                        out_hbm.at[pl.ds(c * N + NS * STRIPE, TAIL)])


def _sc_aggregate(h_stacked, src2, dst):
    mesh = plsc.VectorSubcoreMesh(core_axis_name="c", subcore_axis_name="s")
    k = pl.kernel(
        _sc_agg_body,
        out_type=jax.ShapeDtypeStruct((NC * N, H), jnp.float32),
        mesh=mesh,
        scratch_types=(
            pltpu.VMEM((EPT,), jnp.int32),
            pltpu.VMEM((EPT,), jnp.int32),
            pltpu.VMEM((C,), jnp.int32),
            pltpu.VMEM((C,), jnp.int32),
            pltpu.VMEM((C, H), jnp.float32),
            pltpu.VMEM((C, H), jnp.float32),
            pltpu.VMEM((ETAIL,), jnp.int32),
            pltpu.VMEM((ZROWSA, H), jnp.float32),
            pltpu.VMEM_SHARED((N, H), jnp.float32),
            pltpu.SemaphoreType.DMA,
            pltpu.SemaphoreType.DMA,
        ),
    )
    return k(h_stacked, src2, dst)


CC = 80
E_PER_W = 5040
E_PAD_CNT = E_PER_W * NC * NS
NCHUNK_CNT = E_PER_W // CC


def _sc_cnt_body(dst_hbm, cnt_hbm, dst1d, dstb, onesb, zbuf, cntacc):
    c = lax.axis_index("c")
    s = lax.axis_index("s")

    def _zrow(i, _):
        for j in range(H // 16):
            zbuf[i, pl.ds(j * 16, 16)] = jnp.zeros((16,), jnp.float32)
        return 0
    lax.fori_loop(0, ZROWS, _zrow, 0)

    def _ones(i, _):
        for j in range(H // 16):
            onesb[i, pl.ds(j * 16, 16)] = jnp.ones((16,), jnp.float32)
        return 0
    lax.fori_loop(0, CC, _ones, 0)

    stripe = s * STRIPE
    last = s == NS - 1

    def _zcp(j, _):
        pltpu.sync_copy(zbuf, cntacc.at[pl.ds(stripe + j * ZROWS, ZROWS)])
        return 0
    lax.fori_loop(0, STRIPE // ZROWS, _zcp, 0)

    @pl.when(last)
    def _():
        pltpu.sync_copy(zbuf, cntacc.at[pl.ds(NS * STRIPE, TAIL)])
        pltpu.sync_copy(zbuf, cntacc.at[pl.ds(N, 16)])

    w = c * NS + s
    pltpu.sync_copy(dst_hbm.at[pl.ds(w * E_PER_W, E_PER_W)], dst1d)

    plsc.subcore_barrier()

    def _chunk(t, _):
        for j in range(CC // 16):
            dstb[pl.ds(j * 16, 16)] = dst1d[pl.ds(t * CC + j * 16, 16)]
        pltpu.sync_copy(onesb, cntacc.at[dstb], add=True)
        return 0

    lax.fori_loop(0, NCHUNK_CNT, _chunk, 0)

    plsc.subcore_barrier()

    pltpu.sync_copy(cntacc.at[pl.ds(stripe, STRIPE)],
                    cnt_hbm.at[c, pl.ds(stripe, STRIPE)])

    @pl.when(last)
    def _():
        pltpu.sync_copy(cntacc.at[pl.ds(NS * STRIPE, TAIL)],
                        cnt_hbm.at[c, pl.ds(NS * STRIPE, TAIL)])


def _sc_counts(dst_pad):
    mesh = plsc.VectorSubcoreMesh(core_axis_name="c", subcore_axis_name="s")
    k = pl.kernel(
        _sc_cnt_body,
        out_type=jax.ShapeDtypeStruct((NC, N, H), jnp.float32),
        mesh=mesh,
        scratch_types=(
            pltpu.VMEM((E_PER_W,), jnp.int32),
            pltpu.VMEM((CC,), jnp.int32),
            pltpu.VMEM((CC, H), jnp.float32),
            pltpu.VMEM((ZROWS, H), jnp.float32),
            pltpu.VMEM_SHARED((N + 16, H), jnp.float32),
        ),
    )
    return k(dst_pad)


BT = 1000


def _dense0_body(s_ref, cnt_ref, x_ref, wl_ref, bl_ref, wr_ref,
                 data_ref, w1_ref, b1_ref, w2_ref, b2_ref,
                 hn_ref, st_ref, mlp_ref):
    summed = jnp.concatenate([s_ref[0], s_ref[1]], axis=-1)
    cnt = cnt_ref[0, :, 0:1] + cnt_ref[1, :, 0:1]
    inv = 1.0 / jnp.maximum(cnt, 1.0)
    mean = summed * inv
    out = (jnp.dot(mean, wl_ref[:], preferred_element_type=jnp.float32)
           + bl_ref[0]
           + jnp.dot(x_ref[:], wr_ref[:], preferred_element_type=jnp.float32))
    nrm = jnp.sqrt(jnp.sum(out * out, axis=-1, keepdims=True))
    hn = out / jnp.maximum(nrm, 1e-12)
    hn_ref[:] = hn
    st_ref[0] = hn[:, :H]
    st_ref[1] = hn[:, H:]
    hid = jnp.maximum(
        jnp.dot(data_ref[:], w1_ref[:], preferred_element_type=jnp.float32)
        + b1_ref[0], 0.0)
    mlp_ref[:] = jax.nn.sigmoid(
        jnp.dot(hid, w2_ref[:], preferred_element_type=jnp.float32)
        + b2_ref[0])


def _dense1_body(s_ref, cnt_ref, x_ref, wl_ref, bl_ref, wr_ref, hn_ref):
    summed = jnp.concatenate([s_ref[0], s_ref[1]], axis=-1)
    cnt = cnt_ref[0, :, 0:1] + cnt_ref[1, :, 0:1]
    inv = 1.0 / jnp.maximum(cnt, 1.0)
    mean = summed * inv
    out = (jnp.dot(mean, wl_ref[:], preferred_element_type=jnp.float32)
           + bl_ref[0]
           + jnp.dot(x_ref[:], wr_ref[:], preferred_element_type=jnp.float32))
    nrm = jnp.sqrt(jnp.sum(out * out, axis=-1, keepdims=True))
    hn_ref[:] = out / jnp.maximum(nrm, 1e-12)


def _row_spec(shape):
    nd = len(shape)
    if nd == 2:
        return pl.BlockSpec((BT,) + shape[1:], lambda i: (i,) + (0,) * (nd - 1))
    return pl.BlockSpec((shape[0], BT) + shape[2:], lambda i: (0, i) + (0,) * (nd - 2))


def _full_spec(shape):
    nd = len(shape)
    return pl.BlockSpec(shape, lambda i: (0,) * nd)


def _tc_dense0(summed, cnt, x, wlT, bl, wrT, data, w1T, b1, w2T, b2):
    grid = (N // BT,)
    return pl.pallas_call(
        _dense0_body,
        grid=grid,
        in_specs=[
            _row_spec((2, N, H)), _row_spec((2, N, H)), _row_spec((N, D)),
            _full_spec((D, D)), _full_spec((1, D)), _full_spec((D, D)),
            _row_spec((N, D)), _full_spec((D, 32)), _full_spec((1, 32)),
            _full_spec((32, 2)), _full_spec((1, 2)),
        ],
        out_specs=[_row_spec((N, D)), _row_spec((2, N, H)), _row_spec((N, 2))],
        out_shape=[
            jax.ShapeDtypeStruct((N, D), jnp.float32),
            jax.ShapeDtypeStruct((2, N, H), jnp.float32),
            jax.ShapeDtypeStruct((N, 2), jnp.float32),
        ],
    )(summed, cnt, x, wlT, bl, wrT, data, w1T, b1, w2T, b2)


def _tc_dense1(summed, cnt, h, wlT, bl, wrT):
    grid = (N // BT,)
    return pl.pallas_call(
        _dense1_body,
        grid=grid,
        in_specs=[
            _row_spec((2, N, H)), _row_spec((2, N, H)), _row_spec((N, D)),
            _full_spec((D, D)), _full_spec((1, D)), _full_spec((D, D)),
        ],
        out_specs=[_row_spec((N, D))],
        out_shape=[jax.ShapeDtypeStruct((N, D), jnp.float32)],
    )(summed, cnt, h, wlT, bl, wrT)


def kernel(x, edge_index, data, Wl0, bl0, Wr0, Wl1, bl1, Wr1, W1, b1, W2, b2):
    src = edge_index[0]
    dst = edge_index[1]
    src2 = jnp.concatenate([src, src + N])
    dst_pad = jnp.concatenate(
        [dst, jnp.full((E_PAD_CNT - E,), N, jnp.int32)])
    x_st = jnp.concatenate([x[:, :H], x[:, H:]], axis=0)

    cnt = _sc_counts(dst_pad)
    summed0 = _sc_aggregate(x_st, src2, dst)
    h0, h0_st, mlp_out = _tc_dense0(
        summed0.reshape(2, N, H), cnt, x,
        Wl0.T, bl0.reshape(1, D), Wr0.T,
        data, W1.T, b1.reshape(1, 32), W2.T, b2.reshape(1, 2))

    summed1 = _sc_aggregate(h0_st.reshape(2 * N, H), src2, dst)
    w_pred = _tc_dense1(summed1.reshape(2, N, H), cnt,
                        h0, Wl1.T, bl1.reshape(1, D), Wr1.T)[0]
    return (w_pred, mlp_out)


# speedup vs baseline: 7.1033x; 1.0217x over previous
import jax
import jax.numpy as jnp
from jax import lax
from jax.experimental import pallas as pl
from jax.experimental.pallas import tpu as pltpu
from jax.experimental.pallas import tpu_sc as plsc

N = 10000
E = 160000
D = 256
H = 128
NC = 2
NS = 16
C = 128
EPT = E // NS
NRING = 78
ETAIL = EPT - NRING * C
STRIPE = 624
TAIL = N - NS * STRIPE
ZROWS = 16
ZROWSA = 8


def _sc_agg_body(h_hbm, src2_hbm, dst_hbm, out_hbm,
                 srcb0, srcb1, dstb0, dstb1, rows0, rows1, dstbt,
                 zbuf, acc, gsem0, gsem1, isem0, isem1):
    c = lax.axis_index("c")
    s = lax.axis_index("s")
    srcs, dsts, rows = (srcb0, srcb1), (dstb0, dstb1), (rows0, rows1)
    gsems, isems = (gsem0, gsem1), (isem0, isem1)
    ebase = c * E + s * EPT
    dbase = s * EPT

    def _fire_idx(t, b):
        pltpu.async_copy(src2_hbm.at[pl.ds(ebase + t * C, C)], srcs[b], isems[b])
        pltpu.async_copy(dst_hbm.at[pl.ds(dbase + t * C, C)], dsts[b], isems[b])

    def _wait_idx(b):
        pltpu.make_async_copy(src2_hbm.at[pl.ds(ebase, C)], srcs[b], isems[b]).wait()
        pltpu.make_async_copy(dst_hbm.at[pl.ds(dbase, C)], dsts[b], isems[b]).wait()

    def _fire_gather(b):
        pltpu.async_copy(h_hbm.at[srcs[b]], rows[b], gsems[b])

    def _wait_scatter(b):
        pltpu.make_async_copy(h_hbm.at[srcs[b]], rows[b], gsems[b]).wait()
        pltpu.sync_copy(rows[b], acc.at[dsts[b]], add=True)

    _fire_idx(0, 0)

    def _zrow(i, _):
        for j in range(H // 16):
            zbuf[i, pl.ds(j * 16, 16)] = jnp.zeros((16,), jnp.float32)
        return 0
    lax.fori_loop(0, ZROWSA, _zrow, 0)

    stripe = s * STRIPE
    last = s == NS - 1

    def _zcp(j, _):
        pltpu.sync_copy(zbuf, acc.at[pl.ds(stripe + j * ZROWSA, ZROWSA)])
        return 0
    lax.fori_loop(0, STRIPE // ZROWSA, _zcp, 0)

    @pl.when(last)
    def _():
        pltpu.sync_copy(zbuf, acc.at[pl.ds(NS * STRIPE, ZROWSA)])
        pltpu.sync_copy(zbuf, acc.at[pl.ds(NS * STRIPE + ZROWSA, ZROWSA)])

    plsc.subcore_barrier()

    _wait_idx(0)
    _fire_gather(0)
    _fire_idx(1, 1)

    def _step(t, b):
        _wait_idx(b)
        _fire_gather(b)
        _wait_scatter(1 - b)
        _fire_idx(t + 1, 1 - b)

    def _group(g, _):
        _step(g * 2 + 1, 1)
        _step(g * 2 + 2, 0)
        return 0

    lax.fori_loop(0, (NRING - 2) // 2, _group, 0)

    _wait_idx(1)
    _fire_gather(1)
    _wait_scatter(0)
    _wait_scatter(1)

    tb = NRING * C
    pltpu.async_copy(src2_hbm.at[pl.ds(ebase + tb, ETAIL)],
                     dstbt, isem0)
    pltpu.make_async_copy(src2_hbm.at[pl.ds(ebase + tb, ETAIL)],
                          dstbt, isem0).wait()
    pltpu.async_copy(h_hbm.at[dstbt], rows0.at[pl.ds(0, ETAIL)], gsem0)
    pltpu.make_async_copy(h_hbm.at[dstbt], rows0.at[pl.ds(0, ETAIL)],
                          gsem0).wait()
    pltpu.sync_copy(dst_hbm.at[pl.ds(dbase + tb, ETAIL)], dstbt)
    pltpu.sync_copy(rows0.at[pl.ds(0, ETAIL)], acc.at[dstbt], add=True)

    plsc.subcore_barrier()

    pltpu.sync_copy(acc.at[pl.ds(stripe, STRIPE)],
                    out_hbm.at[pl.ds(c * N + stripe, STRIPE)])

    @pl.when(last)
    def _():
        pltpu.sync_copy(acc.at[pl.ds(NS * STRIPE, TAIL)],
                        out_hbm.at[pl.ds(c * N + NS * STRIPE, TAIL)])


def _sc_aggregate(h_interleaved, src2, dst):
    mesh = plsc.VectorSubcoreMesh(core_axis_name="c", subcore_axis_name="s")
    k = pl.kernel(
        _sc_agg_body,
        out_type=jax.ShapeDtypeStruct((NC * N, H), jnp.float32),
        mesh=mesh,
        scratch_types=(
            pltpu.VMEM((C,), jnp.int32),
            pltpu.VMEM((C,), jnp.int32),
            pltpu.VMEM((C,), jnp.int32),
            pltpu.VMEM((C,), jnp.int32),
            pltpu.VMEM((C, H), jnp.float32),
            pltpu.VMEM((C, H), jnp.float32),
            pltpu.VMEM((ETAIL,), jnp.int32),
            pltpu.VMEM((ZROWSA, H), jnp.float32),
            pltpu.VMEM_SHARED((N, H), jnp.float32),
            pltpu.SemaphoreType.DMA,
            pltpu.SemaphoreType.DMA,
            pltpu.SemaphoreType.DMA,
            pltpu.SemaphoreType.DMA,
        ),
    )
    return k(h_interleaved, src2, dst)


CC = 80
E_PER_W = 5040
E_PAD_CNT = E_PER_W * NC * NS
NCHUNK_CNT = E_PER_W // CC


def _sc_cnt_body(dst_hbm, cnt_hbm, dst1d, dstb, onesb, zbuf, cntacc):
    c = lax.axis_index("c")
    s = lax.axis_index("s")

    def _zrow(i, _):
        for j in range(H // 16):
            zbuf[i, pl.ds(j * 16, 16)] = jnp.zeros((16,), jnp.float32)
        return 0
    lax.fori_loop(0, ZROWS, _zrow, 0)

    def _ones(i, _):
        for j in range(H // 16):
            onesb[i, pl.ds(j * 16, 16)] = jnp.ones((16,), jnp.float32)
        return 0
    lax.fori_loop(0, CC, _ones, 0)

    stripe = s * STRIPE
    last = s == NS - 1

    def _zcp(j, _):
        pltpu.sync_copy(zbuf, cntacc.at[pl.ds(stripe + j * ZROWS, ZROWS)])
        return 0
    lax.fori_loop(0, STRIPE // ZROWS, _zcp, 0)

    @pl.when(last)
    def _():
        pltpu.sync_copy(zbuf, cntacc.at[pl.ds(NS * STRIPE, TAIL)])
        pltpu.sync_copy(zbuf, cntacc.at[pl.ds(N, 16)])

    w = c * NS + s
    pltpu.sync_copy(dst_hbm.at[pl.ds(w * E_PER_W, E_PER_W)], dst1d)

    plsc.subcore_barrier()

    def _chunk(t, _):
        for j in range(CC // 16):
            dstb[pl.ds(j * 16, 16)] = dst1d[pl.ds(t * CC + j * 16, 16)]
        pltpu.sync_copy(onesb, cntacc.at[dstb], add=True)
        return 0

    lax.fori_loop(0, NCHUNK_CNT, _chunk, 0)

    plsc.subcore_barrier()

    pltpu.sync_copy(cntacc.at[pl.ds(stripe, STRIPE)],
                    cnt_hbm.at[c, pl.ds(stripe, STRIPE)])

    @pl.when(last)
    def _():
        pltpu.sync_copy(cntacc.at[pl.ds(NS * STRIPE, TAIL)],
                        cnt_hbm.at[c, pl.ds(NS * STRIPE, TAIL)])


def _sc_counts(dst_pad):
    mesh = plsc.VectorSubcoreMesh(core_axis_name="c", subcore_axis_name="s")
    k = pl.kernel(
        _sc_cnt_body,
        out_type=jax.ShapeDtypeStruct((NC, N, H), jnp.float32),
        mesh=mesh,
        scratch_types=(
            pltpu.VMEM((E_PER_W,), jnp.int32),
            pltpu.VMEM((CC,), jnp.int32),
            pltpu.VMEM((CC, H), jnp.float32),
            pltpu.VMEM((ZROWS, H), jnp.float32),
            pltpu.VMEM_SHARED((N + 16, H), jnp.float32),
        ),
    )
    return k(dst_pad)


BT = 1000


def _dense0_body(s_ref, cnt_ref, x_ref, wl_ref, bl_ref, wr_ref,
                 data_ref, w1_ref, b1_ref, w2_ref, b2_ref,
                 hn_ref, mlp_ref):
    summed = jnp.concatenate([s_ref[0], s_ref[1]], axis=-1)
    cnt = cnt_ref[0, :, 0:1] + cnt_ref[1, :, 0:1]
    inv = 1.0 / jnp.maximum(cnt, 1.0)
    mean = summed * inv
    out = (jnp.dot(mean, wl_ref[:], preferred_element_type=jnp.float32)
           + bl_ref[0]
           + jnp.dot(x_ref[:], wr_ref[:], preferred_element_type=jnp.float32))
    nrm = jnp.sqrt(jnp.sum(out * out, axis=-1, keepdims=True))
    hn_ref[:] = out / jnp.maximum(nrm, 1e-12)
    hid = jnp.maximum(
        jnp.dot(data_ref[:], w1_ref[:], preferred_element_type=jnp.float32)
        + b1_ref[0], 0.0)
    mlp_ref[:] = jax.nn.sigmoid(
        jnp.dot(hid, w2_ref[:], preferred_element_type=jnp.float32)
        + b2_ref[0])


def _dense1_body(s_ref, cnt_ref, x_ref, wl_ref, bl_ref, wr_ref, hn_ref):
    summed = jnp.concatenate([s_ref[0], s_ref[1]], axis=-1)
    cnt = cnt_ref[0, :, 0:1] + cnt_ref[1, :, 0:1]
    inv = 1.0 / jnp.maximum(cnt, 1.0)
    mean = summed * inv
    out = (jnp.dot(mean, wl_ref[:], preferred_element_type=jnp.float32)
           + bl_ref[0]
           + jnp.dot(x_ref[:], wr_ref[:], preferred_element_type=jnp.float32))
    nrm = jnp.sqrt(jnp.sum(out * out, axis=-1, keepdims=True))
    hn_ref[:] = out / jnp.maximum(nrm, 1e-12)


def _row_spec(shape):
    nd = len(shape)
    if nd == 2:
        return pl.BlockSpec((BT,) + shape[1:], lambda i: (i,) + (0,) * (nd - 1))
    return pl.BlockSpec((shape[0], BT) + shape[2:], lambda i: (0, i) + (0,) * (nd - 2))


def _full_spec(shape):
    nd = len(shape)
    return pl.BlockSpec(shape, lambda i: (0,) * nd)


def _tc_dense0(summed, cnt, x, wlT, bl, wrT, data, w1T, b1, w2T, b2):
    grid = (N // BT,)
    return pl.pallas_call(
        _dense0_body,
        grid=grid,
        in_specs=[
            _row_spec((2, N, H)), _row_spec((2, N, H)), _row_spec((N, D)),
            _full_spec((D, D)), _full_spec((1, D)), _full_spec((D, D)),
            _row_spec((N, D)), _full_spec((D, 32)), _full_spec((1, 32)),
            _full_spec((32, 2)), _full_spec((1, 2)),
        ],
        out_specs=[_row_spec((N, D)), _row_spec((N, 2))],
        out_shape=[
            jax.ShapeDtypeStruct((N, D), jnp.float32),
            jax.ShapeDtypeStruct((N, 2), jnp.float32),
        ],
    )(summed, cnt, x, wlT, bl, wrT, data, w1T, b1, w2T, b2)


def _tc_dense1(summed, cnt, h, wlT, bl, wrT):
    grid = (N // BT,)
    return pl.pallas_call(
        _dense1_body,
        grid=grid,
        in_specs=[
            _row_spec((2, N, H)), _row_spec((2, N, H)), _row_spec((N, D)),
            _full_spec((D, D)), _full_spec((1, D)), _full_spec((D, D)),
        ],
        out_specs=[_row_spec((N, D))],
        out_shape=[jax.ShapeDtypeStruct((N, D), jnp.float32)],
    )(summed, cnt, h, wlT, bl, wrT)


def kernel(x, edge_index, data, Wl0, bl0, Wr0, Wl1, bl1, Wr1, W1, b1, W2, b2):
    src = edge_index[0]
    dst = edge_index[1]
    src2 = jnp.concatenate([2 * src, 2 * src + 1])
    dst_pad = jnp.concatenate(
        [dst, jnp.full((E_PAD_CNT - E,), N, jnp.int32)])

    cnt = _sc_counts(dst_pad)
    summed0 = _sc_aggregate(x.reshape(2 * N, H), src2, dst)
    h0, mlp_out = _tc_dense0(
        summed0.reshape(2, N, H), cnt, x,
        Wl0.T, bl0.reshape(1, D), Wr0.T,
        data, W1.T, b1.reshape(1, 32), W2.T, b2.reshape(1, 2))

    summed1 = _sc_aggregate(h0.reshape(2 * N, H), src2, dst)
    w_pred = _tc_dense1(summed1.reshape(2, N, H), cnt,
                        h0, Wl1.T, bl1.reshape(1, D), Wr1.T)[0]
    return (w_pred, mlp_out)
